# baseline (speedup 1.0000x reference)
import functools

import jax
import jax.numpy as jnp
from jax import lax
from jax.experimental import pallas as pl
from jax.experimental.pallas import tpu as pltpu
from jax.experimental.pallas import tpu_sc as plsc

_B, _L, _F, _E, _H = 1024, 200, 32, 128, 128
_T, _P, _MAXF = 2, 512, 64
_EPS = 1e-12
_VLEN = _F + 1
_S = _L + _VLEN
_LANES = 16
_NCH = _E // _LANES
_UNROLL = 4
_G = 2


def _fast_rsqrt(x):
    i = plsc.bitcast(x, jnp.int32)
    y = plsc.bitcast(jnp.int32(0x5F3759DF) - lax.shift_right_logical(i, 1),
                     jnp.float32)
    for _ in range(2):
        y = y * (1.5 - 0.5 * x * y * y)
    return y


def _sc_text(tok_table, idx_t, tt_t, pospre, dseg, ln_g, ln_b):
    info = plsc.get_sparse_core_info()
    nc, ns = info.num_cores, info.num_subcores
    nw = nc * ns
    bw = _B // nw
    mesh = plsc.VectorSubcoreMesh(core_axis_name="c", subcore_axis_name="s")

    @functools.partial(
        pl.kernel,
        mesh=mesh,
        compiler_params=pltpu.CompilerParams(use_tc_tiling_on_sc=False,
                                             needs_layout_passes=False),
        out_type=jax.ShapeDtypeStruct((_S * _B, _E), jnp.float32),
        scratch_types=[
            pltpu.VMEM((_L // _G, _G * bw), jnp.int32),
            pltpu.VMEM((_L // _G, _G * bw), jnp.int32),
            pltpu.VMEM((_G * bw, _E), jnp.float32),
            pltpu.VMEM((_G * bw, _E), jnp.float32),
            pltpu.VMEM((_G * bw, _E), jnp.float32),
            pltpu.VMEM((_G * bw, _E), jnp.float32),
            pltpu.VMEM((_L, _E), jnp.float32),
            pltpu.VMEM((_LANES, _E), jnp.float32),
            pltpu.SemaphoreType.DMA,
            pltpu.SemaphoreType.DMA,
            pltpu.SemaphoreType.DMA,
            pltpu.SemaphoreType.DMA,
            pltpu.SemaphoreType.DMA,
            pltpu.SemaphoreType.DMA,
            pltpu.SemaphoreType.DMA,
            pltpu.SemaphoreType.DMA,
        ],
    )
    def k(table_hbm, idx_hbm, tt_hbm, pospre_hbm, dseg_hbm, g_hbm, b_hbm,
          out_hbm, idx_v, tt_v, rows0, rows1, rows2, rows3, pospre_v, const_v,
          sg0, sg1, sg2, sg3, so0, so1, so2, so3):
        wid = lax.axis_index("s") * nc + lax.axis_index("c")
        bufs = (rows0, rows1, rows2, rows3)
        gsems = (sg0, sg1, sg2, sg3)
        osems = (so0, so1, so2, so3)

        pltpu.sync_copy(idx_hbm.at[wid], idx_v)
        pltpu.sync_copy(tt_hbm.at[wid], tt_v)
        pltpu.sync_copy(pospre_hbm, pospre_v)
        pltpu.sync_copy(dseg_hbm, const_v.at[pl.ds(0, 1)])
        pltpu.sync_copy(g_hbm, const_v.at[pl.ds(1, 1)])
        pltpu.sync_copy(b_hbm, const_v.at[pl.ds(2, 1)])

        dsegs = [const_v[0, pl.ds(c * _LANES, _LANES)] for c in range(_NCH)]
        gs = [const_v[1, pl.ds(c * _LANES, _LANES)] for c in range(_NCH)]
        bs = [const_v[2, pl.ds(c * _LANES, _LANES)] for c in range(_NCH)]

        ng = _L // _G

        def start_gather(g, buf, sem):
            pltpu.async_copy(table_hbm.at[idx_v.at[g]], buf, sem)

        def wait_gather(g, buf, sem):
            pltpu.make_async_copy(table_hbm.at[idx_v.at[g]], buf, sem).wait()

        def out_copies(g, buf, sem, fire):
            for u in range(_G):
                src = buf.at[pl.ds(u * bw, bw)]
                dst = out_hbm.at[pl.ds((g * _G + u) * _B + wid * bw, bw)]
                if fire:
                    pltpu.async_copy(src, dst, sem)
                else:
                    pltpu.make_async_copy(src, dst, sem).wait()

        start_gather(0, bufs[0], gsems[0])
        start_gather(1, bufs[1], gsems[1])

        def seq_body(g, carry):
            p = lax.rem(g, 4)

            def with_buf(p_static):
                buf = bufs[p_static]
                nxt = (p_static + 2) % 4
                nbuf = bufs[nxt]
                @pl.when(g >= 2)
                def _():
                    out_copies(g - 2, nbuf, osems[nxt], fire=False)

                @pl.when(g + 2 < ng)
                def _():
                    start_gather(g + 2, nbuf, gsems[nxt])

                wait_gather(g, buf, gsems[p_static])

                for u in range(_G):
                    l = g * _G + u
                    posl = [pospre_v[l, pl.ds(c * _LANES, _LANES)]
                            for c in range(_NCH)]

                    @plsc.parallel_loop(0, bw, 1, unroll=_UNROLL)
                    def _row(j, u=u, posl=posl):
                        gg = jnp.full((_LANES,), g, jnp.int32)
                        jj = jnp.full((_LANES,), u * bw + j, jnp.int32)
                        tf = plsc.load_gather(tt_v, [gg, jj]).astype(
                            jnp.float32)
                        accs = []
                        for c in range(_NCH):
                            a = (buf[u * bw + j, pl.ds(c * _LANES, _LANES)]
                                 + posl[c] + tf * dsegs[c])
                            accs.append(a)
                        vsum = accs[0]
                        for c in range(1, _NCH):
                            vsum = vsum + accs[c]
                        vsq = accs[0] * accs[0]
                        for c in range(1, _NCH):
                            vsq = vsq + accs[c] * accs[c]
                        s1 = jnp.sum(vsum)
                        s2 = jnp.sum(vsq)
                        mu = s1 * (1.0 / _E)
                        var = s2 * (1.0 / _E) - mu * mu
                        var_v = jnp.full((_LANES,), var + _EPS, jnp.float32)
                        inv_v = _fast_rsqrt(var_v)
                        mu_v = jnp.full((_LANES,), mu, jnp.float32)
                        for c in range(_NCH):
                            o = (accs[c] - mu_v) * inv_v * gs[c] + bs[c]
                            buf[u * bw + j, pl.ds(c * _LANES, _LANES)] = o

                out_copies(g, buf, osems[p_static], fire=True)

            lax.switch(p, (lambda: with_buf(0), lambda: with_buf(1),
                           lambda: with_buf(2), lambda: with_buf(3)))
            return carry

        lax.fori_loop(0, ng, seq_body, 0)
        out_copies(ng - 2, bufs[(ng - 2) % 4], osems[(ng - 2) % 4],
                   fire=False)
        out_copies(ng - 1, bufs[(ng - 1) % 4], osems[(ng - 1) % 4],
                   fire=False)

    return k(tok_table, idx_t, tt_t, pospre, dseg,
             ln_g.reshape(1, _E), ln_b.reshape(1, _E))


def _tc_visual_body(alias_ref, ve_ref, vm_ref, im_ref, vpos_ref, img_ref,
                    vseg_ref, g_ref, b_ref, vg_ref, vb_ref,
                    out_ref, mask_ref):
    del alias_ref
    f32 = jnp.float32
    ve = ve_ref[...]
    vmu = jnp.mean(ve, axis=-1, keepdims=True)
    vvar = jnp.mean(ve * ve, axis=-1, keepdims=True) - vmu * vmu
    ven = (ve - vmu) * lax.rsqrt(vvar + _EPS) * vg_ref[...] + vb_ref[...]
    vpos = vpos_ref[...]
    vseg0 = vseg_ref[...][0]
    vrows = ven + vpos[1:][:, None, :] + vseg0[None, None, :]
    vrow0 = img_ref[...][0] + vpos[0] + vseg0
    bb = ve.shape[1]
    vrow0 = jnp.broadcast_to(vrow0[None, None, :], (1, bb, _E))

    emb = jnp.concatenate([vrow0, vrows], axis=0)
    mu = jnp.mean(emb, axis=-1, keepdims=True)
    var = jnp.mean(emb * emb, axis=-1, keepdims=True) - mu * mu
    normed = (emb - mu) * lax.rsqrt(var + _EPS) * g_ref[...] + b_ref[...]
    pad = jnp.zeros((out_ref.shape[0] - _VLEN, bb, _E), f32)
    out_ref[...] = jnp.concatenate([normed, pad], axis=0)

    vm = vm_ref[...]
    img_mask = (jnp.sum(vm, axis=1, keepdims=True) > 0).astype(vm.dtype)
    mask_ref[...] = jnp.concatenate([im_ref[...], img_mask, vm], axis=1)


def _tc_visual(sc_out, ve_t, visual_mask, input_mask,
               vpos_s, img_table, vseg_table, ln_g, ln_b, vln_g, vln_b):
    bb = 8
    grid = (_B // bb,)
    vblk = 40
    const = lambda *shape: pl.BlockSpec(shape, lambda i: (0,) * len(shape))
    return pl.pallas_call(
        _tc_visual_body,
        grid=grid,
        in_specs=[
            pl.BlockSpec(memory_space=pl.ANY),
            pl.BlockSpec((_F, bb, _E), lambda i: (0, i, 0)),
            pl.BlockSpec((bb, _F), lambda i: (i, 0)),
            pl.BlockSpec((bb, _L), lambda i: (i, 0)),
            const(_VLEN, _E),
            const(1, _E),
            const(1, _E),
            const(1, _E),
            const(1, _E),
            const(1, _E),
            const(1, _E),
        ],
        out_specs=[
            pl.BlockSpec((vblk, bb, _E), lambda i: (_L // vblk, i, 0)),
            pl.BlockSpec((bb, _S), lambda i: (i, 0)),
        ],
        out_shape=[
            jax.ShapeDtypeStruct((_S, _B, _E), jnp.float32),
            jax.ShapeDtypeStruct((_B, _S), jnp.int32),
        ],
        input_output_aliases={0: 0},
    )(sc_out, ve_t, visual_mask, input_mask,
      vpos_s, img_table, vseg_table,
      ln_g.reshape(1, _E), ln_b.reshape(1, _E),
      vln_g.reshape(1, _E), vln_b.reshape(1, _E))


def kernel(input_ids, token_type_ids, input_mask, visual_embeds, visual_mask,
           tok_table, pos_table, seg_table, vpos_table, img_table, vseg_table,
           ln_g, ln_b, vln_g, vln_b):
    info = plsc.get_sparse_core_info()
    nw = info.num_cores * info.num_subcores
    bw = _B // nw
    idx_t = input_ids.reshape(nw, bw, _L).transpose(0, 2, 1).reshape(
        nw, _L // _G, _G * bw)
    tt_t = token_type_ids.reshape(nw, bw, _L).transpose(0, 2, 1).reshape(
        nw, _L // _G, _G * bw)
    pospre = pos_table[:_L] + seg_table[0]
    dseg = (seg_table[1] - seg_table[0]).reshape(1, _E)
    sc_flat = _sc_text(tok_table, idx_t, tt_t, pospre, dseg, ln_g, ln_b)
    sc_out = sc_flat.reshape(_S, _B, _E)
    ve_t = visual_embeds.transpose(1, 0, 2)
    emb_t, mask = _tc_visual(
        sc_out, ve_t, visual_mask, input_mask,
        vpos_table[:_VLEN], img_table, vseg_table, ln_g, ln_b, vln_g, vln_b)
    emb = emb_t.transpose(1, 0, 2)
    return emb, mask


# speedup vs baseline: 1.2306x; 1.2306x over previous
import functools

import jax
import jax.numpy as jnp
from jax import lax
from jax.experimental import pallas as pl
from jax.experimental.pallas import tpu as pltpu
from jax.experimental.pallas import tpu_sc as plsc

_B, _L, _F, _E, _H = 1024, 200, 32, 128, 128
_T, _P, _MAXF = 2, 512, 64
_EPS = 1e-12
_VLEN = _F + 1
_S = _L + _VLEN
_LANES = 16
_NCH = _E // _LANES
_UNROLL = 4
_G = 2


def _fast_rsqrt(x):
    i = plsc.bitcast(x, jnp.int32)
    y = plsc.bitcast(jnp.int32(0x5F3759DF) - lax.shift_right_logical(i, 1),
                     jnp.float32)
    for _ in range(2):
        y = y * (1.5 - 0.5 * x * y * y)
    return y


def _sc_text(tok_table, idx_t, tt_t, pospre, dseg, ln_g, ln_b):
    info = plsc.get_sparse_core_info()
    nc, ns = info.num_cores, info.num_subcores
    nw = nc * ns
    wseq, wbat = 8, nw // 8
    ls = _L // wseq
    bw = _B // wbat
    mesh = plsc.VectorSubcoreMesh(core_axis_name="c", subcore_axis_name="s")

    @functools.partial(
        pl.kernel,
        mesh=mesh,
        compiler_params=pltpu.CompilerParams(use_tc_tiling_on_sc=False,
                                             needs_layout_passes=False),
        out_type=jax.ShapeDtypeStruct((_S * _B, _E), jnp.float32),
        scratch_types=[
            pltpu.VMEM((ls, bw), jnp.int32),
            pltpu.VMEM((ls, bw), jnp.int32),
            pltpu.VMEM((bw, _E), jnp.float32),
            pltpu.VMEM((bw, _E), jnp.float32),
            pltpu.VMEM((bw, _E), jnp.float32),
            pltpu.VMEM((ls, _E), jnp.float32),
            pltpu.VMEM((_LANES, _E), jnp.float32),
            pltpu.SemaphoreType.DMA,
            pltpu.SemaphoreType.DMA,
            pltpu.SemaphoreType.DMA,
            pltpu.SemaphoreType.DMA,
            pltpu.SemaphoreType.DMA,
            pltpu.SemaphoreType.DMA,
        ],
    )
    def k(table_hbm, idx_hbm, tt_hbm, pospre_hbm, dseg_hbm, g_hbm, b_hbm,
          out_hbm, idx_v, tt_v, rows0, rows1, rows2, pospre_v, const_v,
          sg0, sg1, sg2, so0, so1, so2):
        wid = lax.axis_index("s") * nc + lax.axis_index("c")
        ws = lax.div(wid, wbat)
        wb = lax.rem(wid, wbat)
        bufs = (rows0, rows1, rows2)
        gsems = (sg0, sg1, sg2)
        osems = (so0, so1, so2)

        pltpu.sync_copy(idx_hbm.at[wid], idx_v)
        pltpu.sync_copy(tt_hbm.at[wid], tt_v)
        pltpu.sync_copy(pospre_hbm.at[pl.ds(ws * ls, ls)], pospre_v)
        pltpu.sync_copy(dseg_hbm, const_v.at[pl.ds(0, 1)])
        pltpu.sync_copy(g_hbm, const_v.at[pl.ds(1, 1)])
        pltpu.sync_copy(b_hbm, const_v.at[pl.ds(2, 1)])

        dsegs = [const_v[0, pl.ds(c * _LANES, _LANES)] for c in range(_NCH)]
        gs = [const_v[1, pl.ds(c * _LANES, _LANES)] for c in range(_NCH)]
        bs = [const_v[2, pl.ds(c * _LANES, _LANES)] for c in range(_NCH)]

        def start_gather(s, buf, sem):
            for h in range(bw // 128):
                pltpu.async_copy(
                    table_hbm.at[idx_v.at[s, pl.ds(h * 128, 128)]],
                    buf.at[pl.ds(h * 128, 128)], sem)

        def wait_gather(s, buf, sem):
            for h in range(bw // 128):
                pltpu.make_async_copy(
                    table_hbm.at[idx_v.at[s, pl.ds(h * 128, 128)]],
                    buf.at[pl.ds(h * 128, 128)], sem).wait()

        def out_copy(s, buf, sem, fire):
            dst = out_hbm.at[pl.ds((ws * ls + s) * _B + wb * bw, bw)]
            if fire:
                pltpu.async_copy(buf, dst, sem)
            else:
                pltpu.make_async_copy(buf, dst, sem).wait()

        start_gather(0, bufs[0], gsems[0])

        def seq_body(s, carry):
            p = lax.rem(s, 3)

            def with_buf(p_static):
                buf = bufs[p_static]
                nxt = (p_static + 1) % 3
                nbuf = bufs[nxt]
                @pl.when(s >= 2)
                def _():
                    out_copy(s - 2, nbuf, osems[nxt], fire=False)

                @pl.when(s + 1 < ls)
                def _():
                    start_gather(s + 1, nbuf, gsems[nxt])

                wait_gather(s, buf, gsems[p_static])

                posl = [pospre_v[s, pl.ds(c * _LANES, _LANES)]
                        for c in range(_NCH)]

                @plsc.parallel_loop(0, bw, 1, unroll=_UNROLL)
                def _row(j):
                    ss = jnp.full((_LANES,), s, jnp.int32)
                    jj = jnp.full((_LANES,), j, jnp.int32)
                    tf = plsc.load_gather(tt_v, [ss, jj]).astype(jnp.float32)
                    accs = []
                    for c in range(_NCH):
                        a = (buf[j, pl.ds(c * _LANES, _LANES)]
                             + posl[c] + tf * dsegs[c])
                        accs.append(a)
                    vsum = accs[0]
                    for c in range(1, _NCH):
                        vsum = vsum + accs[c]
                    vsq = accs[0] * accs[0]
                    for c in range(1, _NCH):
                        vsq = vsq + accs[c] * accs[c]
                    s1 = jnp.sum(vsum)
                    s2 = jnp.sum(vsq)
                    mu = s1 * (1.0 / _E)
                    var = s2 * (1.0 / _E) - mu * mu
                    var_v = jnp.full((_LANES,), var + _EPS, jnp.float32)
                    inv_v = _fast_rsqrt(var_v)
                    mu_v = jnp.full((_LANES,), mu, jnp.float32)
                    for c in range(_NCH):
                        o = (accs[c] - mu_v) * inv_v * gs[c] + bs[c]
                        buf[j, pl.ds(c * _LANES, _LANES)] = o

                out_copy(s, buf, osems[p_static], fire=True)

            lax.switch(p, (lambda: with_buf(0), lambda: with_buf(1),
                           lambda: with_buf(2)))
            return carry

        lax.fori_loop(0, ls, seq_body, 0)
        out_copy(ls - 2, bufs[(ls - 2) % 3], osems[(ls - 2) % 3], fire=False)
        out_copy(ls - 1, bufs[(ls - 1) % 3], osems[(ls - 1) % 3], fire=False)

    return k(tok_table, idx_t, tt_t, pospre, dseg,
             ln_g.reshape(1, _E), ln_b.reshape(1, _E))


def _tc_visual_body(alias_ref, ve_ref, vm_ref, im_ref, vpos_ref, img_ref,
                    vseg_ref, g_ref, b_ref, vg_ref, vb_ref,
                    out_ref, mask_ref):
    del alias_ref
    f32 = jnp.float32
    ve = ve_ref[...]
    vmu = jnp.mean(ve, axis=-1, keepdims=True)
    vvar = jnp.mean(ve * ve, axis=-1, keepdims=True) - vmu * vmu
    ven = (ve - vmu) * lax.rsqrt(vvar + _EPS) * vg_ref[...] + vb_ref[...]
    vpos = vpos_ref[...]
    vseg0 = vseg_ref[...][0]
    vrows = ven + vpos[1:][:, None, :] + vseg0[None, None, :]
    vrow0 = img_ref[...][0] + vpos[0] + vseg0
    bb = ve.shape[1]
    vrow0 = jnp.broadcast_to(vrow0[None, None, :], (1, bb, _E))

    emb = jnp.concatenate([vrow0, vrows], axis=0)
    mu = jnp.mean(emb, axis=-1, keepdims=True)
    var = jnp.mean(emb * emb, axis=-1, keepdims=True) - mu * mu
    normed = (emb - mu) * lax.rsqrt(var + _EPS) * g_ref[...] + b_ref[...]
    pad = jnp.zeros((out_ref.shape[0] - _VLEN, bb, _E), f32)
    out_ref[...] = jnp.concatenate([normed, pad], axis=0)

    vm = vm_ref[...]
    img_mask = (jnp.sum(vm, axis=1, keepdims=True) > 0).astype(vm.dtype)
    mask_ref[...] = jnp.concatenate([im_ref[...], img_mask, vm], axis=1)


def _tc_visual(sc_out, ve_t, visual_mask, input_mask,
               vpos_s, img_table, vseg_table, ln_g, ln_b, vln_g, vln_b):
    bb = 8
    grid = (_B // bb,)
    vblk = 40
    const = lambda *shape: pl.BlockSpec(shape, lambda i: (0,) * len(shape))
    return pl.pallas_call(
        _tc_visual_body,
        grid=grid,
        in_specs=[
            pl.BlockSpec(memory_space=pl.ANY),
            pl.BlockSpec((_F, bb, _E), lambda i: (0, i, 0)),
            pl.BlockSpec((bb, _F), lambda i: (i, 0)),
            pl.BlockSpec((bb, _L), lambda i: (i, 0)),
            const(_VLEN, _E),
            const(1, _E),
            const(1, _E),
            const(1, _E),
            const(1, _E),
            const(1, _E),
            const(1, _E),
        ],
        out_specs=[
            pl.BlockSpec((vblk, bb, _E), lambda i: (_L // vblk, i, 0)),
            pl.BlockSpec((bb, _S), lambda i: (i, 0)),
        ],
        out_shape=[
            jax.ShapeDtypeStruct((_S, _B, _E), jnp.float32),
            jax.ShapeDtypeStruct((_B, _S), jnp.int32),
        ],
        input_output_aliases={0: 0},
    )(sc_out, ve_t, visual_mask, input_mask,
      vpos_s, img_table, vseg_table,
      ln_g.reshape(1, _E), ln_b.reshape(1, _E),
      vln_g.reshape(1, _E), vln_b.reshape(1, _E))


def kernel(input_ids, token_type_ids, input_mask, visual_embeds, visual_mask,
           tok_table, pos_table, seg_table, vpos_table, img_table, vseg_table,
           ln_g, ln_b, vln_g, vln_b):
    info = plsc.get_sparse_core_info()
    nw = info.num_cores * info.num_subcores
    wseq, wbat = 8, nw // 8
    ls, bw = _L // wseq, _B // wbat

    def _arrange(a):
        return (a.T.reshape(wseq, ls, wbat, bw)
                .transpose(0, 2, 1, 3).reshape(nw, ls, bw))

    idx_t = _arrange(input_ids)
    tt_t = _arrange(token_type_ids)
    pospre = pos_table[:_L] + seg_table[0]
    dseg = (seg_table[1] - seg_table[0]).reshape(1, _E)
    sc_flat = _sc_text(tok_table, idx_t, tt_t, pospre, dseg, ln_g, ln_b)
    sc_out = sc_flat.reshape(_S, _B, _E)
    ve_t = visual_embeds.transpose(1, 0, 2)
    emb_t, mask = _tc_visual(
        sc_out, ve_t, visual_mask, input_mask,
        vpos_table[:_VLEN], img_table, vseg_table, ln_g, ln_b, vln_g, vln_b)
    emb = emb_t.transpose(1, 0, 2)
    return emb, mask


# speedup vs baseline: 1.8356x; 1.4917x over previous
import functools

import jax
import jax.numpy as jnp
from jax import lax
from jax.experimental import pallas as pl
from jax.experimental.pallas import tpu as pltpu
from jax.experimental.pallas import tpu_sc as plsc

_B, _L, _F, _E, _H = 1024, 200, 32, 128, 128
_T, _P, _MAXF = 2, 512, 64
_EPS = 1e-12
_VLEN = _F + 1
_S = _L + _VLEN


def _sc_gather_t(tok_table, idx_t):
    info = plsc.get_sparse_core_info()
    nc, ns = info.num_cores, info.num_subcores
    nw = nc * ns
    wseq, wbat = 8, nw // 8
    ls = _L // wseq
    bw = _B // wbat
    mesh = plsc.VectorSubcoreMesh(core_axis_name="c", subcore_axis_name="s")

    @functools.partial(
        pl.kernel,
        mesh=mesh,
        compiler_params=pltpu.CompilerParams(use_tc_tiling_on_sc=False,
                                             needs_layout_passes=False),
        out_type=jax.ShapeDtypeStruct((_L * _B, _E), jnp.float32),
        scratch_types=[
            pltpu.VMEM((ls, bw), jnp.int32),
            pltpu.VMEM((bw, _E), jnp.float32),
            pltpu.VMEM((bw, _E), jnp.float32),
            pltpu.VMEM((bw, _E), jnp.float32),
            pltpu.SemaphoreType.DMA,
            pltpu.SemaphoreType.DMA,
            pltpu.SemaphoreType.DMA,
            pltpu.SemaphoreType.DMA,
            pltpu.SemaphoreType.DMA,
            pltpu.SemaphoreType.DMA,
        ],
    )
    def k(table_hbm, idx_hbm, out_hbm, idx_v, rows0, rows1, rows2,
          sg0, sg1, sg2, so0, so1, so2):
        wid = lax.axis_index("s") * nc + lax.axis_index("c")
        ws = lax.div(wid, wbat)
        wb = lax.rem(wid, wbat)
        bufs = (rows0, rows1, rows2)
        gsems = (sg0, sg1, sg2)
        osems = (so0, so1, so2)

        pltpu.sync_copy(idx_hbm.at[wid], idx_v)

        def start_gather(s, buf, sem):
            for h in range(bw // 128):
                pltpu.async_copy(
                    table_hbm.at[idx_v.at[s, pl.ds(h * 128, 128)]],
                    buf.at[pl.ds(h * 128, 128)], sem)

        def wait_gather(s, buf, sem):
            for h in range(bw // 128):
                pltpu.make_async_copy(
                    table_hbm.at[idx_v.at[s, pl.ds(h * 128, 128)]],
                    buf.at[pl.ds(h * 128, 128)], sem).wait()

        def out_copy(s, buf, sem, fire):
            dst = out_hbm.at[pl.ds((ws * ls + s) * _B + wb * bw, bw)]
            if fire:
                pltpu.async_copy(buf, dst, sem)
            else:
                pltpu.make_async_copy(buf, dst, sem).wait()

        start_gather(0, bufs[0], gsems[0])

        def seq_body(s, carry):
            p = lax.rem(s, 3)

            def with_buf(p_static):
                buf = bufs[p_static]
                nxt = (p_static + 1) % 3
                nbuf = bufs[nxt]
                @pl.when(s >= 2)
                def _():
                    out_copy(s - 2, nbuf, osems[nxt], fire=False)

                @pl.when(s + 1 < ls)
                def _():
                    start_gather(s + 1, nbuf, gsems[nxt])

                wait_gather(s, buf, gsems[p_static])
                out_copy(s, buf, osems[p_static], fire=True)

            lax.switch(p, (lambda: with_buf(0), lambda: with_buf(1),
                           lambda: with_buf(2)))
            return carry

        lax.fori_loop(0, ls, seq_body, 0)
        out_copy(ls - 2, bufs[(ls - 2) % 3], osems[(ls - 2) % 3], fire=False)
        out_copy(ls - 1, bufs[(ls - 1) % 3], osems[(ls - 1) % 3], fire=False)

    return k(tok_table, idx_t)


def _tc_full_body(tok_ref, tt_ref, im_ref, ve_ref, vm_ref, pos_ref, seg_ref,
                  vpos_ref, img_ref, vseg_ref, g_ref, b_ref, vg_ref, vb_ref,
                  out_ref, mask_ref):
    tok = tok_ref[...]
    ttf = jnp.transpose(tt_ref[...].astype(jnp.float32))[..., None]
    seg = seg_ref[...]
    pos = pos_ref[...]
    text = (tok + pos[:, None, :] + seg[0][None, None, :]
            + ttf * (seg[1] - seg[0])[None, None, :])

    ve = ve_ref[...]
    vmu = jnp.mean(ve, axis=-1, keepdims=True)
    vvar = jnp.mean(ve * ve, axis=-1, keepdims=True) - vmu * vmu
    ven = (ve - vmu) * lax.rsqrt(vvar + _EPS) * vg_ref[...] + vb_ref[...]
    vpos = vpos_ref[...]
    vseg0 = vseg_ref[...][0]
    vrows = ven + vpos[1:][:, None, :] + vseg0[None, None, :]
    vrow0 = img_ref[...][0] + vpos[0] + vseg0
    bb = ve.shape[1]
    vrow0 = jnp.broadcast_to(vrow0[None, None, :], (1, bb, _E))

    emb = jnp.concatenate([text, vrow0, vrows], axis=0)
    mu = jnp.mean(emb, axis=-1, keepdims=True)
    var = jnp.mean(emb * emb, axis=-1, keepdims=True) - mu * mu
    out_ref[...] = (emb - mu) * lax.rsqrt(var + _EPS) * g_ref[...] + b_ref[...]

    vm = vm_ref[...]
    img_mask = (jnp.sum(vm, axis=1, keepdims=True) > 0).astype(vm.dtype)
    mask_ref[...] = jnp.concatenate([im_ref[...], img_mask, vm], axis=1)


def _tc_full(tok_t3, tt_t, input_mask, ve_t, visual_mask,
             pos_s, seg_table, vpos_s, img_table, vseg_table,
             ln_g, ln_b, vln_g, vln_b):
    bb = 8
    grid = (_B // bb,)
    const = lambda *shape: pl.BlockSpec(shape, lambda i: (0,) * len(shape))
    return pl.pallas_call(
        _tc_full_body,
        grid=grid,
        in_specs=[
            pl.BlockSpec((_L, bb, _E), lambda i: (0, i, 0)),
            pl.BlockSpec((bb, _L), lambda i: (i, 0)),
            pl.BlockSpec((bb, _L), lambda i: (i, 0)),
            pl.BlockSpec((_F, bb, _E), lambda i: (0, i, 0)),
            pl.BlockSpec((bb, _F), lambda i: (i, 0)),
            const(_L, _E),
            const(_T, _E),
            const(_VLEN, _E),
            const(1, _E),
            const(1, _E),
            const(1, _E),
            const(1, _E),
            const(1, _E),
            const(1, _E),
        ],
        out_specs=[
            pl.BlockSpec((_S, bb, _E), lambda i: (0, i, 0)),
            pl.BlockSpec((bb, _S), lambda i: (i, 0)),
        ],
        out_shape=[
            jax.ShapeDtypeStruct((_S, _B, _E), jnp.float32),
            jax.ShapeDtypeStruct((_B, _S), jnp.int32),
        ],
    )(tok_t3, tt_t, input_mask, ve_t, visual_mask,
      pos_s, seg_table, vpos_s, img_table, vseg_table,
      ln_g.reshape(1, _E), ln_b.reshape(1, _E),
      vln_g.reshape(1, _E), vln_b.reshape(1, _E))


def kernel(input_ids, token_type_ids, input_mask, visual_embeds, visual_mask,
           tok_table, pos_table, seg_table, vpos_table, img_table, vseg_table,
           ln_g, ln_b, vln_g, vln_b):
    info = plsc.get_sparse_core_info()
    nw = info.num_cores * info.num_subcores
    wseq, wbat = 8, nw // 8
    ls, bw = _L // wseq, _B // wbat

    idx_t = (input_ids.T.reshape(wseq, ls, wbat, bw)
             .transpose(0, 2, 1, 3).reshape(nw, ls, bw))
    tok_t3 = _sc_gather_t(tok_table, idx_t).reshape(_L, _B, _E)
    ve_t = visual_embeds.transpose(1, 0, 2)
    emb_t, mask = _tc_full(
        tok_t3, token_type_ids, input_mask, ve_t, visual_mask,
        pos_table[:_L], seg_table, vpos_table[:_VLEN], img_table, vseg_table,
        ln_g, ln_b, vln_g, vln_b)
    emb = emb_t.transpose(1, 0, 2)
    return emb, mask


# speedup vs baseline: 2.1254x; 1.1579x over previous
import functools

import jax
import jax.numpy as jnp
from jax import lax
from jax.experimental import pallas as pl
from jax.experimental.pallas import tpu as pltpu
from jax.experimental.pallas import tpu_sc as plsc

_B, _L, _F, _E, _H = 1024, 200, 32, 128, 128
_T, _P, _MAXF = 2, 512, 64
_EPS = 1e-12
_VLEN = _F + 1
_S = _L + _VLEN


def _sc_gather_t(tok_table, idx_t):
    info = plsc.get_sparse_core_info()
    nc, ns = info.num_cores, info.num_subcores
    nw = nc * ns
    wseq, wbat = 8, nw // 8
    ls = _L // wseq
    bw = _B // wbat
    mesh = plsc.VectorSubcoreMesh(core_axis_name="c", subcore_axis_name="s")

    @functools.partial(
        pl.kernel,
        mesh=mesh,
        compiler_params=pltpu.CompilerParams(use_tc_tiling_on_sc=False,
                                             needs_layout_passes=False),
        out_type=jax.ShapeDtypeStruct((_L * _B, _E), jnp.float32),
        scratch_types=[
            pltpu.VMEM((ls, bw), jnp.int32),
            pltpu.VMEM((bw, _E), jnp.float32),
            pltpu.VMEM((bw, _E), jnp.float32),
            pltpu.VMEM((bw, _E), jnp.float32),
            pltpu.SemaphoreType.DMA,
            pltpu.SemaphoreType.DMA,
            pltpu.SemaphoreType.DMA,
            pltpu.SemaphoreType.DMA,
            pltpu.SemaphoreType.DMA,
            pltpu.SemaphoreType.DMA,
        ],
    )
    def k(table_hbm, idx_hbm, out_hbm, idx_v, rows0, rows1, rows2,
          sg0, sg1, sg2, so0, so1, so2):
        wid = lax.axis_index("s") * nc + lax.axis_index("c")
        ws = lax.div(wid, wbat)
        wb = lax.rem(wid, wbat)
        bufs = (rows0, rows1, rows2)
        gsems = (sg0, sg1, sg2)
        osems = (so0, so1, so2)

        pltpu.sync_copy(idx_hbm.at[wid], idx_v)

        def start_gather(s, buf, sem):
            for h in range(bw // 128):
                pltpu.async_copy(
                    table_hbm.at[idx_v.at[s, pl.ds(h * 128, 128)]],
                    buf.at[pl.ds(h * 128, 128)], sem)

        def wait_gather(s, buf, sem):
            for h in range(bw // 128):
                pltpu.make_async_copy(
                    table_hbm.at[idx_v.at[s, pl.ds(h * 128, 128)]],
                    buf.at[pl.ds(h * 128, 128)], sem).wait()

        def out_copy(s, buf, sem, fire):
            dst = out_hbm.at[pl.ds((ws * ls + s) * _B + wb * bw, bw)]
            if fire:
                pltpu.async_copy(buf, dst, sem)
            else:
                pltpu.make_async_copy(buf, dst, sem).wait()

        start_gather(0, bufs[0], gsems[0])

        def seq_body(s, carry):
            p = lax.rem(s, 3)

            def with_buf(p_static):
                buf = bufs[p_static]
                nxt = (p_static + 1) % 3
                nbuf = bufs[nxt]
                @pl.when(s >= 2)
                def _():
                    out_copy(s - 2, nbuf, osems[nxt], fire=False)

                @pl.when(s + 1 < ls)
                def _():
                    start_gather(s + 1, nbuf, gsems[nxt])

                wait_gather(s, buf, gsems[p_static])
                out_copy(s, buf, osems[p_static], fire=True)

            lax.switch(p, (lambda: with_buf(0), lambda: with_buf(1),
                           lambda: with_buf(2)))
            return carry

        lax.fori_loop(0, ls, seq_body, 0)
        out_copy(ls - 2, bufs[(ls - 2) % 3], osems[(ls - 2) % 3], fire=False)
        out_copy(ls - 1, bufs[(ls - 1) % 3], osems[(ls - 1) % 3], fire=False)

    return k(tok_table, idx_t)


def _tc_full_body(tok_ref, tt_ref, im_ref, ve_ref, vm_ref, pos_ref, seg_ref,
                  vpos_ref, img_ref, vseg_ref, g_ref, b_ref, vg_ref, vb_ref,
                  out_ref, mask_ref):
    tok = tok_ref[...]
    ttf = jnp.transpose(tt_ref[...].astype(jnp.float32))[..., None]
    seg = seg_ref[...]
    pos = pos_ref[...]
    text = (tok + pos[:, None, :] + seg[0][None, None, :]
            + ttf * (seg[1] - seg[0])[None, None, :])

    ve = ve_ref[...]
    vmu = jnp.mean(ve, axis=-1, keepdims=True)
    vvar = jnp.mean(ve * ve, axis=-1, keepdims=True) - vmu * vmu
    ven = (ve - vmu) * lax.rsqrt(vvar + _EPS) * vg_ref[...] + vb_ref[...]
    vpos = vpos_ref[...]
    vseg0 = vseg_ref[...][0]
    vrows = ven + vpos[1:][:, None, :] + vseg0[None, None, :]
    vrow0 = img_ref[...][0] + vpos[0] + vseg0
    bb = ve.shape[1]
    vrow0 = jnp.broadcast_to(vrow0[None, None, :], (1, bb, _E))

    emb = jnp.concatenate([text, vrow0, vrows], axis=0)
    mu = jnp.mean(emb, axis=-1, keepdims=True)
    var = jnp.mean(emb * emb, axis=-1, keepdims=True) - mu * mu
    out_ref[...] = (emb - mu) * lax.rsqrt(var + _EPS) * g_ref[...] + b_ref[...]

    vm = vm_ref[...]
    img_mask = (jnp.sum(vm, axis=1, keepdims=True) > 0).astype(vm.dtype)
    mask_ref[...] = jnp.concatenate([im_ref[...], img_mask, vm], axis=1)


def _tc_full(tok_t3, tt_t, input_mask, ve_t, visual_mask,
             pos_s, seg_table, vpos_s, img_table, vseg_table,
             ln_g, ln_b, vln_g, vln_b):
    bb = 16
    grid = (_B // bb,)
    const = lambda *shape: pl.BlockSpec(shape, lambda i: (0,) * len(shape))
    return pl.pallas_call(
        _tc_full_body,
        grid=grid,
        in_specs=[
            pl.BlockSpec((_L, bb, _E), lambda i: (0, i, 0)),
            pl.BlockSpec((bb, _L), lambda i: (i, 0)),
            pl.BlockSpec((bb, _L), lambda i: (i, 0)),
            pl.BlockSpec((_F, bb, _E), lambda i: (0, i, 0)),
            pl.BlockSpec((bb, _F), lambda i: (i, 0)),
            const(_L, _E),
            const(_T, _E),
            const(_VLEN, _E),
            const(1, _E),
            const(1, _E),
            const(1, _E),
            const(1, _E),
            const(1, _E),
            const(1, _E),
        ],
        out_specs=[
            pl.BlockSpec((_S, bb, _E), lambda i: (0, i, 0)),
            pl.BlockSpec((bb, _S), lambda i: (i, 0)),
        ],
        out_shape=[
            jax.ShapeDtypeStruct((_S, _B, _E), jnp.float32),
            jax.ShapeDtypeStruct((_B, _S), jnp.int32),
        ],
    )(tok_t3, tt_t, input_mask, ve_t, visual_mask,
      pos_s, seg_table, vpos_s, img_table, vseg_table,
      ln_g.reshape(1, _E), ln_b.reshape(1, _E),
      vln_g.reshape(1, _E), vln_b.reshape(1, _E))


def kernel(input_ids, token_type_ids, input_mask, visual_embeds, visual_mask,
           tok_table, pos_table, seg_table, vpos_table, img_table, vseg_table,
           ln_g, ln_b, vln_g, vln_b):
    info = plsc.get_sparse_core_info()
    nw = info.num_cores * info.num_subcores
    wseq, wbat = 8, nw // 8
    ls, bw = _L // wseq, _B // wbat

    idx_t = (input_ids.T.reshape(wseq, ls, wbat, bw)
             .transpose(0, 2, 1, 3).reshape(nw, ls, bw))
    tok_t3 = _sc_gather_t(tok_table, idx_t).reshape(_L, _B, _E)
    ve_t = visual_embeds.transpose(1, 0, 2)
    emb_t, mask = _tc_full(
        tok_t3, token_type_ids, input_mask, ve_t, visual_mask,
        pos_table[:_L], seg_table, vpos_table[:_VLEN], img_table, vseg_table,
        ln_g, ln_b, vln_g, vln_b)
    emb = emb_t.transpose(1, 0, 2)
    return emb, mask


# speedup vs baseline: 2.2432x; 1.0554x over previous
import functools

import jax
import jax.numpy as jnp
from jax import lax
from jax.experimental import pallas as pl
from jax.experimental.pallas import tpu as pltpu
from jax.experimental.pallas import tpu_sc as plsc

_B, _L, _F, _E, _H = 1024, 200, 32, 128, 128
_T, _P, _MAXF = 2, 512, 64
_EPS = 1e-12
_VLEN = _F + 1
_S = _L + _VLEN


def _sc_gather_t(tok_table, idx_t):
    info = plsc.get_sparse_core_info()
    nc, ns = info.num_cores, info.num_subcores
    nw = nc * ns
    wseq, wbat = 8, nw // 8
    ls = _L // wseq
    bw = _B // wbat
    mesh = plsc.VectorSubcoreMesh(core_axis_name="c", subcore_axis_name="s")

    @functools.partial(
        pl.kernel,
        mesh=mesh,
        compiler_params=pltpu.CompilerParams(use_tc_tiling_on_sc=False,
                                             needs_layout_passes=False),
        out_type=jax.ShapeDtypeStruct((_L * _B, _E), jnp.float32),
        scratch_types=[
            pltpu.VMEM((ls, bw), jnp.int32),
            pltpu.VMEM((bw, _E), jnp.float32),
            pltpu.VMEM((bw, _E), jnp.float32),
            pltpu.VMEM((bw, _E), jnp.float32),
            pltpu.SemaphoreType.DMA,
            pltpu.SemaphoreType.DMA,
            pltpu.SemaphoreType.DMA,
            pltpu.SemaphoreType.DMA,
            pltpu.SemaphoreType.DMA,
            pltpu.SemaphoreType.DMA,
        ],
    )
    def k(table_hbm, idx_hbm, out_hbm, idx_v, rows0, rows1, rows2,
          sg0, sg1, sg2, so0, so1, so2):
        wid = lax.axis_index("s") * nc + lax.axis_index("c")
        ws = lax.div(wid, wbat)
        wb = lax.rem(wid, wbat)
        bufs = (rows0, rows1, rows2)
        gsems = (sg0, sg1, sg2)
        osems = (so0, so1, so2)

        pltpu.sync_copy(idx_hbm.at[wid], idx_v)

        def start_gather(s, buf, sem):
            for h in range(bw // 128):
                pltpu.async_copy(
                    table_hbm.at[idx_v.at[s, pl.ds(h * 128, 128)]],
                    buf.at[pl.ds(h * 128, 128)], sem)

        def wait_gather(s, buf, sem):
            for h in range(bw // 128):
                pltpu.make_async_copy(
                    table_hbm.at[idx_v.at[s, pl.ds(h * 128, 128)]],
                    buf.at[pl.ds(h * 128, 128)], sem).wait()

        def out_copy(s, buf, sem, fire):
            dst = out_hbm.at[pl.ds((ws * ls + s) * _B + wb * bw, bw)]
            if fire:
                pltpu.async_copy(buf, dst, sem)
            else:
                pltpu.make_async_copy(buf, dst, sem).wait()

        start_gather(0, bufs[0], gsems[0])

        def seq_body(s, carry):
            p = lax.rem(s, 3)

            def with_buf(p_static):
                buf = bufs[p_static]
                nxt = (p_static + 1) % 3
                nbuf = bufs[nxt]
                @pl.when(s >= 2)
                def _():
                    out_copy(s - 2, nbuf, osems[nxt], fire=False)

                @pl.when(s + 1 < ls)
                def _():
                    start_gather(s + 1, nbuf, gsems[nxt])

                wait_gather(s, buf, gsems[p_static])
                out_copy(s, buf, osems[p_static], fire=True)

            lax.switch(p, (lambda: with_buf(0), lambda: with_buf(1),
                           lambda: with_buf(2)))
            return carry

        lax.fori_loop(0, ls, seq_body, 0)
        out_copy(ls - 2, bufs[(ls - 2) % 3], osems[(ls - 2) % 3], fire=False)
        out_copy(ls - 1, bufs[(ls - 1) % 3], osems[(ls - 1) % 3], fire=False)

    return k(tok_table, idx_t)


def _tc_full_body(tok_ref, tt_ref, im_ref, ve_ref, vm_ref, pos_ref, seg_ref,
                  vpos_ref, img_ref, vseg_ref, g_ref, b_ref, vg_ref, vb_ref,
                  out_ref, mask_ref):
    tok = tok_ref[...]
    ttf = jnp.transpose(tt_ref[...].astype(jnp.float32))[..., None]
    seg = seg_ref[...]
    pos = pos_ref[...]
    text = (tok + pos[:, None, :] + seg[0][None, None, :]
            + ttf * (seg[1] - seg[0])[None, None, :])

    ve = ve_ref[...]
    vmu = jnp.mean(ve, axis=-1, keepdims=True)
    vvar = jnp.mean(ve * ve, axis=-1, keepdims=True) - vmu * vmu
    ven = (ve - vmu) * lax.rsqrt(vvar + _EPS) * vg_ref[...] + vb_ref[...]
    vpos = vpos_ref[...]
    vseg0 = vseg_ref[...][0]
    vrows = ven + vpos[1:][:, None, :] + vseg0[None, None, :]
    vrow0 = img_ref[...][0] + vpos[0] + vseg0
    bb = ve.shape[1]
    vrow0 = jnp.broadcast_to(vrow0[None, None, :], (1, bb, _E))

    emb = jnp.concatenate([text, vrow0, vrows], axis=0)
    mu = jnp.mean(emb, axis=-1, keepdims=True)
    var = jnp.mean(emb * emb, axis=-1, keepdims=True) - mu * mu
    out_ref[...] = (emb - mu) * lax.rsqrt(var + _EPS) * g_ref[...] + b_ref[...]

    vm = vm_ref[...]
    img_mask = (jnp.sum(vm, axis=1, keepdims=True) > 0).astype(vm.dtype)
    mask_ref[...] = jnp.concatenate([im_ref[...], img_mask, vm], axis=1)


def _tc_full(tok_t3, tt_t, input_mask, ve_t, visual_mask,
             pos_s, seg_table, vpos_s, img_table, vseg_table,
             ln_g, ln_b, vln_g, vln_b):
    bb = 32
    grid = (_B // bb,)
    const = lambda *shape: pl.BlockSpec(shape, lambda i: (0,) * len(shape))
    return pl.pallas_call(
        _tc_full_body,
        grid=grid,
        in_specs=[
            pl.BlockSpec((_L, bb, _E), lambda i: (0, i, 0)),
            pl.BlockSpec((bb, _L), lambda i: (i, 0)),
            pl.BlockSpec((bb, _L), lambda i: (i, 0)),
            pl.BlockSpec((_F, bb, _E), lambda i: (0, i, 0)),
            pl.BlockSpec((bb, _F), lambda i: (i, 0)),
            const(_L, _E),
            const(_T, _E),
            const(_VLEN, _E),
            const(1, _E),
            const(1, _E),
            const(1, _E),
            const(1, _E),
            const(1, _E),
            const(1, _E),
        ],
        out_specs=[
            pl.BlockSpec((_S, bb, _E), lambda i: (0, i, 0)),
            pl.BlockSpec((bb, _S), lambda i: (i, 0)),
        ],
        out_shape=[
            jax.ShapeDtypeStruct((_S, _B, _E), jnp.float32),
            jax.ShapeDtypeStruct((_B, _S), jnp.int32),
        ],
    )(tok_t3, tt_t, input_mask, ve_t, visual_mask,
      pos_s, seg_table, vpos_s, img_table, vseg_table,
      ln_g.reshape(1, _E), ln_b.reshape(1, _E),
      vln_g.reshape(1, _E), vln_b.reshape(1, _E))


def kernel(input_ids, token_type_ids, input_mask, visual_embeds, visual_mask,
           tok_table, pos_table, seg_table, vpos_table, img_table, vseg_table,
           ln_g, ln_b, vln_g, vln_b):
    info = plsc.get_sparse_core_info()
    nw = info.num_cores * info.num_subcores
    wseq, wbat = 8, nw // 8
    ls, bw = _L // wseq, _B // wbat

    idx_t = (input_ids.T.reshape(wseq, ls, wbat, bw)
             .transpose(0, 2, 1, 3).reshape(nw, ls, bw))
    tok_t3 = _sc_gather_t(tok_table, idx_t).reshape(_L, _B, _E)
    ve_t = visual_embeds.transpose(1, 0, 2)
    emb_t, mask = _tc_full(
        tok_t3, token_type_ids, input_mask, ve_t, visual_mask,
        pos_table[:_L], seg_table, vpos_table[:_VLEN], img_table, vseg_table,
        ln_g, ln_b, vln_g, vln_b)
    emb = emb_t.transpose(1, 0, 2)
    return emb, mask
